# baseline (speedup 1.0000x reference)
import functools

import jax
import jax.numpy as jnp
from jax import lax
from jax.experimental import pallas as pl
from jax.experimental.pallas import tpu as pltpu
from jax.experimental.pallas import tpu_sc as plsc

F = 26
V = 100000
K = 16
B = 4096
BF = B * F

NC = 2
NS = 16
NW = NC * NS
RPW = BF // NW
EPW = RPW * K
NCH = 4
CH = EPW // NCH


def _sc_gather_body(eidx_hbm, rid_hbm, emb_hbm, lin_hbm, emb_out, lin_out,
                    eidx_v, vals_v, rid_v, lin_v, sem):
    wid = lax.axis_index("s") * NC + lax.axis_index("c")
    ebase = pl.multiple_of(wid * EPW, EPW)
    for c in range(NCH):
        off = ebase + c * CH
        pltpu.sync_copy(eidx_hbm.at[pl.ds(off, CH)], eidx_v)
        pltpu.async_copy(emb_hbm.at[eidx_v], vals_v, sem).wait()
        pltpu.sync_copy(vals_v, emb_out.at[pl.ds(off, CH)])
    rbase = pl.multiple_of(wid * RPW, RPW)
    pltpu.sync_copy(rid_hbm.at[pl.ds(rbase, RPW)], rid_v)
    pltpu.async_copy(lin_hbm.at[rid_v], lin_v, sem).wait()
    pltpu.sync_copy(lin_v, lin_out.at[pl.ds(rbase, RPW)])


@functools.cache
def _sc_gather():
    return pl.kernel(
        _sc_gather_body,
        out_type=(
            jax.ShapeDtypeStruct((BF * K,), jnp.float32),
            jax.ShapeDtypeStruct((BF,), jnp.float32),
        ),
        mesh=plsc.VectorSubcoreMesh(core_axis_name="c", subcore_axis_name="s"),
        scratch_types=[
            pltpu.VMEM((CH,), jnp.int32),
            pltpu.VMEM((CH,), jnp.float32),
            pltpu.VMEM((RPW,), jnp.int32),
            pltpu.VMEM((RPW,), jnp.float32),
            pltpu.SemaphoreType.DMA,
        ],
    )


def _tc_dense_body(flat_ref, lin_ref, linb_ref, w1_ref, b1_ref, w2_ref,
                   b2_ref, w3_ref, b3_ref, out_ref):
    x = flat_ref[...]
    s = x[:, 0:K]
    for f in range(1, F):
        s = s + x[:, f * K:(f + 1) * K]
    sum_of_square = jnp.sum(s * s)
    square_of_sum = jnp.sum(x * x)
    interaction = 0.5 * (sum_of_square - square_of_sum)
    lin = lin_ref[...]
    line_out = jnp.sum(lin, axis=1, keepdims=True) + linb_ref[...]
    h = jnp.dot(x, w1_ref[...], preferred_element_type=jnp.float32)
    h = jnp.maximum(h + b1_ref[...], 0.0)
    h = jnp.dot(h, w2_ref[...], preferred_element_type=jnp.float32)
    h = jnp.maximum(h + b2_ref[...], 0.0)
    fnn = jnp.dot(h, w3_ref[...], preferred_element_type=jnp.float32)
    fnn = fnn + b3_ref[...]
    out_ref[...] = fnn + line_out + interaction


_tc_dense = pl.pallas_call(
    _tc_dense_body,
    out_shape=jax.ShapeDtypeStruct((B, 1), jnp.float32),
)


def kernel(inputs, emb_table, lin_table, lin_bias, W1, b1, W2, b2, W3, b3):
    rid = (inputs + jnp.arange(F, dtype=jnp.int32)[None, :] * V).reshape(BF)
    eidx = (rid[:, None] * K + jnp.arange(K, dtype=jnp.int32)[None, :]
            ).reshape(BF * K)
    emb1d = emb_table.reshape(F * V * K)
    lin1d = lin_table.reshape(F * V)
    emb_vals, lin_vals = _sc_gather()(eidx, rid, emb1d, lin1d)
    flat = emb_vals.reshape(B, F * K)
    lin_bf = lin_vals.reshape(B, F)
    return _tc_dense(flat, lin_bf, lin_bias, W1, b1, W2, b2, W3, b3)
